# baseline (speedup 1.0000x reference)
import functools

import jax
import jax.numpy as jnp
import numpy as np
from jax import lax
from jax.experimental import pallas as pl
from jax.experimental.pallas import tpu as pltpu
from jax.experimental.pallas import tpu_sc as plsc

K = 10
NG = 8
N = 4096
R = 128
C = 256
NCH = N // C
T = N // R
D = 64
MASKV = np.float32(1e10)
INF = np.float32(np.inf)

_SC_NC = 2
_SC_NW = 32
_B = K * N
_BPW = _B // _SC_NW
_GCH = _BPW // 128


def _make_knn(F):
    def body(clo_ref, chi_ref, frows_ref, fall_ref, sqr_ref, sqc_ref,
             br_ref, bc_ref, idx_ref, dist_ref):
        t = pl.program_id(0)
        lo = clo_ref[t]
        hi = chi_ref[t]
        rows = frows_ref[...]
        sqr = sqr_ref[...]
        br = br_ref[...]

        def fill_chunk(c, carry):
            fc = fall_ref[pl.ds(c * C, C), :]
            sqc = sqc_ref[:, pl.ds(c * C, C)]
            bc = bc_ref[:, pl.ds(c * C, C)]
            d = sqr + sqc - 2.0 * lax.dot_general(
                rows, fc, (((1,), (1,)), ((), ())),
                preferred_element_type=jnp.float32)
            d = jnp.where(br != bc, MASKV, d)
            dist_ref[:, pl.ds(c * C, C)] = d
            return carry

        iota128 = lax.broadcasted_iota(jnp.int32, (R, 128), 1)
        BIGI = np.int32(2 ** 30)
        NSUB = C // 128

        def run(lo, hi):
            lax.fori_loop(lo, hi, fill_chunk, 0)
            args = []
            lastm = None
            prev = None
            for k in range(K):
                def fold_chunk(c, carry, prev=prev):
                    fv, fi = carry
                    for h in range(NSUB):
                        off = c * C + h * 128
                        dv = dist_ref[:, pl.ds(off, 128)]
                        di = off + iota128
                        if prev is not None:
                            dv = jnp.where(di == prev, INF, dv)
                            dist_ref[:, pl.ds(off, 128)] = dv
                        better = dv < fv
                        fv = jnp.where(better, dv, fv)
                        fi = jnp.where(better, di, fi)
                    return fv, fi

                init = (jnp.full((R, 128), np.inf, jnp.float32),
                        jnp.full((R, 128), BIGI, jnp.int32))
                fv, fi = lax.fori_loop(lo, hi, fold_chunk, init)
                lastm = jnp.min(fv, axis=1, keepdims=True)
                a = jnp.min(jnp.where(fv == lastm, fi, BIGI),
                            axis=1, keepdims=True)
                args.append(a)
                prev = a
            return jnp.concatenate(args, axis=1), lastm

        span_idx, span_last = run(lo, hi)
        need_full = jnp.max(span_last) >= MASKV
        idx_ref[...] = lax.cond(need_full,
                                lambda: run(0, NCH)[0],
                                lambda: span_idx)

    grid_spec = pltpu.PrefetchScalarGridSpec(
        num_scalar_prefetch=2,
        grid=(T,),
        in_specs=[
            pl.BlockSpec((R, F), lambda t, *_: (t, 0)),
            pl.BlockSpec((N, F), lambda t, *_: (0, 0)),
            pl.BlockSpec((R, 1), lambda t, *_: (t, 0)),
            pl.BlockSpec((1, N), lambda t, *_: (0, 0)),
            pl.BlockSpec((R, 1), lambda t, *_: (t, 0)),
            pl.BlockSpec((1, N), lambda t, *_: (0, 0)),
        ],
        out_specs=pl.BlockSpec((R, K), lambda t, *_: (t, 0)),
        scratch_shapes=[pltpu.VMEM((R, N), jnp.float32)],
    )
    return pl.pallas_call(
        body, grid_spec=grid_spec,
        out_shape=jax.ShapeDtypeStruct((N, K), jnp.int32))


_KNN = {8: _make_knn(8), 64: _make_knn(64)}


def _lrelu(v):
    return jnp.where(v >= 0, v, 0.01 * v)


def _sq_body(f_ref, sq_ref):
    f = f_ref[...]
    sq_ref[...] = jnp.sum(f * f, axis=1, keepdims=True)


_SQ8 = pl.pallas_call(
    _sq_body,
    grid=(N // 512,),
    in_specs=[pl.BlockSpec((512, 8), lambda i: (i, 0))],
    out_specs=pl.BlockSpec((512, 1), lambda i: (i, 0)),
    out_shape=jax.ShapeDtypeStruct((N, 1), jnp.float32))


_RA = 512


def _make_agg(F_in, F_g):
    def body(f_ref, g_ref, w_ref, b_ref, out_ref, sq_ref):
        xi = f_ref[...]
        acc = jnp.zeros((_RA, D), jnp.float32)
        for k in range(K):
            xj = g_ref[:, k * F_g:k * F_g + F_in]
            e = jnp.concatenate([xi, xj - xi], axis=1)
            h = jnp.dot(e, w_ref[...],
                        preferred_element_type=jnp.float32) + b_ref[...]
            acc = acc + _lrelu(h)
        out_ref[...] = acc
        sq_ref[...] = jnp.sum(acc * acc, axis=1, keepdims=True)

    return pl.pallas_call(
        body,
        grid=(N // _RA,),
        in_specs=[
            pl.BlockSpec((_RA, F_in), lambda i: (i, 0)),
            pl.BlockSpec((_RA, K * F_g), lambda i: (i, 0)),
            pl.BlockSpec((2 * F_in, D), lambda i: (0, 0)),
            pl.BlockSpec((1, D), lambda i: (0, 0)),
        ],
        out_specs=[
            pl.BlockSpec((_RA, D), lambda i: (i, 0)),
            pl.BlockSpec((_RA, 1), lambda i: (i, 0)),
        ],
        out_shape=[
            jax.ShapeDtypeStruct((N, D), jnp.float32),
            jax.ShapeDtypeStruct((N, 1), jnp.float32),
        ])


_AGG1 = _make_agg(8, 16)
_AGG2 = _make_agg(D, D)


@functools.cache
def _build_sc_gather(Dg):
    mesh = plsc.VectorSubcoreMesh(core_axis_name="c", subcore_axis_name="s")

    @functools.partial(
        pl.kernel, mesh=mesh,
        compiler_params=pltpu.CompilerParams(use_tc_tiling_on_sc=False),
        out_type=jax.ShapeDtypeStruct((_B, Dg), jnp.float32),
        scratch_types=[
            pltpu.VMEM((_GCH, 128), jnp.int32),
            pltpu.VMEM((_BPW, Dg), jnp.float32),
            pltpu.SemaphoreType.DMA,
        ])
    def _sc_gather(table_hbm, idx_hbm, out_hbm, idx_v, rows_v, sem):
        wid = lax.axis_index("s") * _SC_NC + lax.axis_index("c")
        base = wid * _BPW
        pltpu.sync_copy(idx_hbm.at[wid], idx_v)
        handles = []
        for j in range(_GCH):
            handles.append(pltpu.async_copy(
                table_hbm.at[idx_v.at[j]], rows_v.at[pl.ds(j * 128, 128)],
                sem))
        for h in handles:
            h.wait()
        pltpu.sync_copy(rows_v, out_hbm.at[pl.ds(base, _BPW)])

    return _sc_gather


def _gather_rows(table, flat_idx):
    idx3 = flat_idx.reshape(_SC_NW, _GCH, 128)
    return _build_sc_gather(table.shape[1])(table, idx3)


_RH = 512


def _head_body(x1, x2, x3, x4, x5, x6, w1_ref, b1_ref, w2_ref, b2_ref,
               br_ref, pooled_ref):
    t = pl.program_id(0)
    xs = (x1, x2, x3, x4, x5, x6)
    u = b1_ref[...] + jnp.zeros((_RH, 512), jnp.float32)
    for i, xr in enumerate(xs):
        u = u + jnp.dot(xr[...], w1_ref[i * D:(i + 1) * D, :],
                        preferred_element_type=jnp.float32)
    h = jnp.dot(_lrelu(u), w2_ref[...],
                preferred_element_type=jnp.float32) + b2_ref[...]
    br = br_ref[...]
    parts = []
    for g in range(NG):
        hg = jnp.where(br == g, h, -INF)
        parts.append(jnp.max(hg, axis=0, keepdims=True))
    tile_max = jnp.concatenate(parts, axis=0)

    @pl.when(t == 0)
    def _():
        pooled_ref[...] = jnp.full((NG, 512), -np.inf, jnp.float32)

    pooled_ref[...] = jnp.maximum(pooled_ref[...], tile_max)


_HEAD = pl.pallas_call(
    _head_body,
    grid=(N // _RH,),
    in_specs=[pl.BlockSpec((_RH, D), lambda i: (i, 0)) for _ in range(6)] + [
        pl.BlockSpec((6 * D, 512), lambda i: (0, 0)),
        pl.BlockSpec((1, 512), lambda i: (0, 0)),
        pl.BlockSpec((512, 512), lambda i: (0, 0)),
        pl.BlockSpec((1, 512), lambda i: (0, 0)),
        pl.BlockSpec((_RH, 1), lambda i: (i, 0)),
    ],
    out_specs=pl.BlockSpec((NG, 512), lambda i: (0, 0)),
    out_shape=jax.ShapeDtypeStruct((NG, 512), jnp.float32))


def _final_body(pooled_ref, w1_ref, b1_ref, w2_ref, b2_ref, out_ref):
    u = jnp.dot(pooled_ref[...], w1_ref[...],
                preferred_element_type=jnp.float32) + b1_ref[...]
    out_ref[...] = jnp.dot(_lrelu(u), w2_ref[...],
                           preferred_element_type=jnp.float32) + b2_ref[...]


_FINAL = pl.pallas_call(
    _final_body,
    out_shape=jax.ShapeDtypeStruct((NG, K), jnp.float32))


def kernel(x, pos, tq, batch, W1, b1, W2, b2,
           l1W1, l1b1, l1W2, l1b2, mW1, mb1, mW2, mb2):
    batch = batch.astype(jnp.int32)
    xx = jnp.concatenate([tq, x, pos], axis=1)
    xx = jnp.pad(xx, ((0, 0), (0, 3)))
    br = batch.reshape(N, 1)
    bc = batch.reshape(1, N)

    g = jnp.arange(NG, dtype=jnp.int32)
    gstart = jnp.searchsorted(batch, g, side="left").astype(jnp.int32)
    gend = jnp.searchsorted(batch, g, side="right").astype(jnp.int32)
    tidx = jnp.arange(T, dtype=jnp.int32)
    tfirst = batch[tidx * R]
    tlast = batch[tidx * R + (R - 1)]
    clo = (gstart[tfirst] // C).astype(jnp.int32)
    chi = ((gend[tlast] + (C - 1)) // C).astype(jnp.int32)

    W1p = jnp.zeros((16, D), jnp.float32)
    W1p = W1p.at[0:5].set(W1[:5]).at[8:13].set(W1[5:])

    f = xx
    sq = _SQ8(f)
    feats = []
    for i in range(6):
        F = 8 if i == 0 else D
        idx = _KNN[F](clo, chi, f, f, sq, sq.reshape(1, N), br, bc)
        flat = idx.reshape(-1)
        if i == 0:
            table = jnp.pad(f, ((0, 0), (0, 8)))
            G = _gather_rows(table, flat).reshape(N, K * 16)
            f, sq = _AGG1(f, G, W1p, b1.reshape(1, D))
        else:
            G = _gather_rows(f, flat).reshape(N, K * D)
            f, sq = _AGG2(f, G, W2, b2.reshape(1, D))
        feats.append(f)

    pooled = _HEAD(*feats, l1W1, l1b1.reshape(1, 512), l1W2,
                   l1b2.reshape(1, 512), br)
    out = _FINAL(pooled, mW1, mb1.reshape(1, 256), mW2, mb2.reshape(1, K))
    return out


# speedup vs baseline: 1.1079x; 1.1079x over previous
import functools

import jax
import jax.numpy as jnp
import numpy as np
from jax import lax
from jax.experimental import pallas as pl
from jax.experimental.pallas import tpu as pltpu
from jax.experimental.pallas import tpu_sc as plsc

K = 10
NG = 8
N = 4096
R = 256
C = 128
NCH = N // C
T = N // R
D = 64
MASKV = np.float32(1e10)
INF = np.float32(np.inf)

_SC_NC = 2
_SC_NW = 32
_B = K * N
_BPW = _B // _SC_NW
_GCH = _BPW // 128


def _make_knn(F):
    def body(clo_ref, chi_ref, frows_ref, fall_ref, sqr_ref, sqc_ref,
             br_ref, bc_ref, idx_ref, dist_ref):
        t = pl.program_id(0)
        lo = clo_ref[t]
        hi = chi_ref[t]
        rows = frows_ref[...]
        sqr = sqr_ref[...]
        br = br_ref[...]

        def fill_chunk(c, carry):
            fc = fall_ref[pl.ds(c * C, C), :]
            sqc = sqc_ref[:, pl.ds(c * C, C)]
            bc = bc_ref[:, pl.ds(c * C, C)]
            d = sqr + sqc - 2.0 * lax.dot_general(
                rows, fc, (((1,), (1,)), ((), ())),
                preferred_element_type=jnp.float32)
            d = jnp.where(br != bc, MASKV, d)
            dist_ref[:, pl.ds(c * C, C)] = d
            return carry

        iota128 = lax.broadcasted_iota(jnp.int32, (R, 128), 1)
        BIGI = np.int32(2 ** 30)
        NSUB = C // 128

        def run(lo, hi):
            lax.fori_loop(lo, hi, fill_chunk, 0)
            args = []
            lastm = None
            prev = None
            for k in range(K):
                def fold_chunk(c, carry, prev=prev):
                    fv, fi = carry
                    for h in range(NSUB):
                        off = c * C + h * 128
                        dv = dist_ref[:, pl.ds(off, 128)]
                        di = off + iota128
                        if prev is not None:
                            dv = jnp.where(di == prev, INF, dv)
                            dist_ref[:, pl.ds(off, 128)] = dv
                        better = dv < fv
                        fv = jnp.where(better, dv, fv)
                        fi = jnp.where(better, di, fi)
                    return fv, fi

                init = (jnp.full((R, 128), np.inf, jnp.float32),
                        jnp.full((R, 128), BIGI, jnp.int32))
                fv, fi = lax.fori_loop(lo, hi, fold_chunk, init)
                lastm = jnp.min(fv, axis=1, keepdims=True)
                a = jnp.min(jnp.where(fv == lastm, fi, BIGI),
                            axis=1, keepdims=True)
                args.append(a)
                prev = a
            return jnp.concatenate(args, axis=1), lastm

        span_idx, span_last = run(lo, hi)
        need_full = jnp.max(span_last) >= MASKV
        idx_ref[...] = lax.cond(need_full,
                                lambda: run(0, NCH)[0],
                                lambda: span_idx)

    grid_spec = pltpu.PrefetchScalarGridSpec(
        num_scalar_prefetch=2,
        grid=(T,),
        in_specs=[
            pl.BlockSpec((R, F), lambda t, *_: (t, 0)),
            pl.BlockSpec((N, F), lambda t, *_: (0, 0)),
            pl.BlockSpec((R, 1), lambda t, *_: (t, 0)),
            pl.BlockSpec((1, N), lambda t, *_: (0, 0)),
            pl.BlockSpec((R, 1), lambda t, *_: (t, 0)),
            pl.BlockSpec((1, N), lambda t, *_: (0, 0)),
        ],
        out_specs=pl.BlockSpec((R, K), lambda t, *_: (t, 0)),
        scratch_shapes=[pltpu.VMEM((R, N), jnp.float32)],
    )
    return pl.pallas_call(
        body, grid_spec=grid_spec,
        out_shape=jax.ShapeDtypeStruct((N, K), jnp.int32))


_KNN = {8: _make_knn(8), 64: _make_knn(64)}


def _lrelu(v):
    return jnp.where(v >= 0, v, 0.01 * v)


def _sq_body(f_ref, sq_ref):
    f = f_ref[...]
    sq_ref[...] = jnp.sum(f * f, axis=1, keepdims=True)


_SQ8 = pl.pallas_call(
    _sq_body,
    grid=(N // 512,),
    in_specs=[pl.BlockSpec((512, 8), lambda i: (i, 0))],
    out_specs=pl.BlockSpec((512, 1), lambda i: (i, 0)),
    out_shape=jax.ShapeDtypeStruct((N, 1), jnp.float32))


_RA = 512


def _make_agg(F_in, F_g):
    def body(f_ref, g_ref, w_ref, b_ref, out_ref, sq_ref):
        xi = f_ref[...]
        acc = jnp.zeros((_RA, D), jnp.float32)
        for k in range(K):
            xj = g_ref[:, k * F_g:k * F_g + F_in]
            e = jnp.concatenate([xi, xj - xi], axis=1)
            h = jnp.dot(e, w_ref[...],
                        preferred_element_type=jnp.float32) + b_ref[...]
            acc = acc + _lrelu(h)
        out_ref[...] = acc
        sq_ref[...] = jnp.sum(acc * acc, axis=1, keepdims=True)

    return pl.pallas_call(
        body,
        grid=(N // _RA,),
        in_specs=[
            pl.BlockSpec((_RA, F_in), lambda i: (i, 0)),
            pl.BlockSpec((_RA, K * F_g), lambda i: (i, 0)),
            pl.BlockSpec((2 * F_in, D), lambda i: (0, 0)),
            pl.BlockSpec((1, D), lambda i: (0, 0)),
        ],
        out_specs=[
            pl.BlockSpec((_RA, D), lambda i: (i, 0)),
            pl.BlockSpec((_RA, 1), lambda i: (i, 0)),
        ],
        out_shape=[
            jax.ShapeDtypeStruct((N, D), jnp.float32),
            jax.ShapeDtypeStruct((N, 1), jnp.float32),
        ])


_AGG1 = _make_agg(8, 16)
_AGG2 = _make_agg(D, D)


@functools.cache
def _build_sc_gather(Dg):
    mesh = plsc.VectorSubcoreMesh(core_axis_name="c", subcore_axis_name="s")

    @functools.partial(
        pl.kernel, mesh=mesh,
        compiler_params=pltpu.CompilerParams(use_tc_tiling_on_sc=False),
        out_type=jax.ShapeDtypeStruct((_B, Dg), jnp.float32),
        scratch_types=[
            pltpu.VMEM((_GCH, 128), jnp.int32),
            pltpu.VMEM((_BPW, Dg), jnp.float32),
            pltpu.SemaphoreType.DMA,
        ])
    def _sc_gather(table_hbm, idx_hbm, out_hbm, idx_v, rows_v, sem):
        wid = lax.axis_index("s") * _SC_NC + lax.axis_index("c")
        base = wid * _BPW
        pltpu.sync_copy(idx_hbm.at[wid], idx_v)
        handles = []
        for j in range(_GCH):
            handles.append(pltpu.async_copy(
                table_hbm.at[idx_v.at[j]], rows_v.at[pl.ds(j * 128, 128)],
                sem))
        for h in handles:
            h.wait()
        pltpu.sync_copy(rows_v, out_hbm.at[pl.ds(base, _BPW)])

    return _sc_gather


def _gather_rows(table, flat_idx):
    idx3 = flat_idx.reshape(_SC_NW, _GCH, 128)
    return _build_sc_gather(table.shape[1])(table, idx3)


_RH = 512


def _head_body(x1, x2, x3, x4, x5, x6, w1_ref, b1_ref, w2_ref, b2_ref,
               br_ref, pooled_ref):
    t = pl.program_id(0)
    xs = (x1, x2, x3, x4, x5, x6)
    u = b1_ref[...] + jnp.zeros((_RH, 512), jnp.float32)
    for i, xr in enumerate(xs):
        u = u + jnp.dot(xr[...], w1_ref[i * D:(i + 1) * D, :],
                        preferred_element_type=jnp.float32)
    h = jnp.dot(_lrelu(u), w2_ref[...],
                preferred_element_type=jnp.float32) + b2_ref[...]
    br = br_ref[...]
    parts = []
    for g in range(NG):
        hg = jnp.where(br == g, h, -INF)
        parts.append(jnp.max(hg, axis=0, keepdims=True))
    tile_max = jnp.concatenate(parts, axis=0)

    @pl.when(t == 0)
    def _():
        pooled_ref[...] = jnp.full((NG, 512), -np.inf, jnp.float32)

    pooled_ref[...] = jnp.maximum(pooled_ref[...], tile_max)


_HEAD = pl.pallas_call(
    _head_body,
    grid=(N // _RH,),
    in_specs=[pl.BlockSpec((_RH, D), lambda i: (i, 0)) for _ in range(6)] + [
        pl.BlockSpec((6 * D, 512), lambda i: (0, 0)),
        pl.BlockSpec((1, 512), lambda i: (0, 0)),
        pl.BlockSpec((512, 512), lambda i: (0, 0)),
        pl.BlockSpec((1, 512), lambda i: (0, 0)),
        pl.BlockSpec((_RH, 1), lambda i: (i, 0)),
    ],
    out_specs=pl.BlockSpec((NG, 512), lambda i: (0, 0)),
    out_shape=jax.ShapeDtypeStruct((NG, 512), jnp.float32))


def _final_body(pooled_ref, w1_ref, b1_ref, w2_ref, b2_ref, out_ref):
    u = jnp.dot(pooled_ref[...], w1_ref[...],
                preferred_element_type=jnp.float32) + b1_ref[...]
    out_ref[...] = jnp.dot(_lrelu(u), w2_ref[...],
                           preferred_element_type=jnp.float32) + b2_ref[...]


_FINAL = pl.pallas_call(
    _final_body,
    out_shape=jax.ShapeDtypeStruct((NG, K), jnp.float32))


def kernel(x, pos, tq, batch, W1, b1, W2, b2,
           l1W1, l1b1, l1W2, l1b2, mW1, mb1, mW2, mb2):
    batch = batch.astype(jnp.int32)
    xx = jnp.concatenate([tq, x, pos], axis=1)
    xx = jnp.pad(xx, ((0, 0), (0, 3)))
    br = batch.reshape(N, 1)
    bc = batch.reshape(1, N)

    g = jnp.arange(NG, dtype=jnp.int32)
    gstart = jnp.searchsorted(batch, g, side="left").astype(jnp.int32)
    gend = jnp.searchsorted(batch, g, side="right").astype(jnp.int32)
    tidx = jnp.arange(T, dtype=jnp.int32)
    tfirst = batch[tidx * R]
    tlast = batch[tidx * R + (R - 1)]
    clo = (gstart[tfirst] // C).astype(jnp.int32)
    chi = ((gend[tlast] + (C - 1)) // C).astype(jnp.int32)

    W1p = jnp.zeros((16, D), jnp.float32)
    W1p = W1p.at[0:5].set(W1[:5]).at[8:13].set(W1[5:])

    f = xx
    sq = _SQ8(f)
    feats = []
    for i in range(6):
        F = 8 if i == 0 else D
        idx = _KNN[F](clo, chi, f, f, sq, sq.reshape(1, N), br, bc)
        flat = idx.reshape(-1)
        if i == 0:
            table = jnp.pad(f, ((0, 0), (0, 8)))
            G = _gather_rows(table, flat).reshape(N, K * 16)
            f, sq = _AGG1(f, G, W1p, b1.reshape(1, D))
        else:
            G = _gather_rows(f, flat).reshape(N, K * D)
            f, sq = _AGG2(f, G, W2, b2.reshape(1, D))
        feats.append(f)

    pooled = _HEAD(*feats, l1W1, l1b1.reshape(1, 512), l1W2,
                   l1b2.reshape(1, 512), br)
    out = _FINAL(pooled, mW1, mb1.reshape(1, 256), mW2, mb2.reshape(1, K))
    return out


# speedup vs baseline: 1.1248x; 1.0152x over previous
import functools

import jax
import jax.numpy as jnp
import numpy as np
from jax import lax
from jax.experimental import pallas as pl
from jax.experimental.pallas import tpu as pltpu
from jax.experimental.pallas import tpu_sc as plsc

K = 10
NG = 8
N = 4096
R = 256
C = 256
NCH = N // C
T = N // R
D = 64
MASKV = np.float32(1e10)
INF = np.float32(np.inf)

_SC_NC = 2
_SC_NW = 32
_B = K * N
_BPW = _B // _SC_NW
_GCH = _BPW // 128


def _make_knn(F):
    def body(clo_ref, chi_ref, frows_ref, fall_ref, sqr_ref, sqc_ref,
             br_ref, bc_ref, idx_ref, dist_ref):
        t = pl.program_id(0)
        lo = clo_ref[t]
        hi = chi_ref[t]
        rows = frows_ref[...]
        sqr = sqr_ref[...]
        br = br_ref[...]

        def fill_chunk(c, carry):
            fc = fall_ref[pl.ds(c * C, C), :]
            sqc = sqc_ref[:, pl.ds(c * C, C)]
            bc = bc_ref[:, pl.ds(c * C, C)]
            d = sqr + sqc - 2.0 * lax.dot_general(
                rows, fc, (((1,), (1,)), ((), ())),
                preferred_element_type=jnp.float32)
            d = jnp.where(br != bc, MASKV, d)
            dist_ref[:, pl.ds(c * C, C)] = d
            return carry

        iota128 = lax.broadcasted_iota(jnp.int32, (R, 128), 1)
        BIGI = np.int32(2 ** 30)
        NSUB = C // 128

        def run(lo, hi):
            lax.fori_loop(lo, hi, fill_chunk, 0)
            args = []
            lastm = None
            prev = None
            for k in range(K):
                def fold_chunk(c, carry, prev=prev):
                    fv, fi = carry
                    for h in range(NSUB):
                        off = c * C + h * 128
                        dv = dist_ref[:, pl.ds(off, 128)]
                        di = off + iota128
                        if prev is not None:
                            dv = jnp.where(di == prev, INF, dv)
                            dist_ref[:, pl.ds(off, 128)] = dv
                        better = dv < fv
                        fv = jnp.where(better, dv, fv)
                        fi = jnp.where(better, di, fi)
                    return fv, fi

                init = (jnp.full((R, 128), np.inf, jnp.float32),
                        jnp.full((R, 128), BIGI, jnp.int32))
                fv, fi = lax.fori_loop(lo, hi, fold_chunk, init)
                lastm = jnp.min(fv, axis=1, keepdims=True)
                a = jnp.min(jnp.where(fv == lastm, fi, BIGI),
                            axis=1, keepdims=True)
                args.append(a)
                prev = a
            return jnp.concatenate(args, axis=1), lastm

        span_idx, span_last = run(lo, hi)
        need_full = jnp.max(span_last) >= MASKV
        idx_ref[...] = lax.cond(need_full,
                                lambda: run(0, NCH)[0],
                                lambda: span_idx)

    grid_spec = pltpu.PrefetchScalarGridSpec(
        num_scalar_prefetch=2,
        grid=(T,),
        in_specs=[
            pl.BlockSpec((R, F), lambda t, *_: (t, 0)),
            pl.BlockSpec((N, F), lambda t, *_: (0, 0)),
            pl.BlockSpec((R, 1), lambda t, *_: (t, 0)),
            pl.BlockSpec((1, N), lambda t, *_: (0, 0)),
            pl.BlockSpec((R, 1), lambda t, *_: (t, 0)),
            pl.BlockSpec((1, N), lambda t, *_: (0, 0)),
        ],
        out_specs=pl.BlockSpec((R, K), lambda t, *_: (t, 0)),
        scratch_shapes=[pltpu.VMEM((R, N), jnp.float32)],
    )
    return pl.pallas_call(
        body, grid_spec=grid_spec,
        out_shape=jax.ShapeDtypeStruct((N, K), jnp.int32))


_KNN = {8: _make_knn(8), 64: _make_knn(64)}


def _lrelu(v):
    return jnp.where(v >= 0, v, 0.01 * v)


def _sq_body(f_ref, sq_ref):
    f = f_ref[...]
    sq_ref[...] = jnp.sum(f * f, axis=1, keepdims=True)


_SQ8 = pl.pallas_call(
    _sq_body,
    grid=(N // 512,),
    in_specs=[pl.BlockSpec((512, 8), lambda i: (i, 0))],
    out_specs=pl.BlockSpec((512, 1), lambda i: (i, 0)),
    out_shape=jax.ShapeDtypeStruct((N, 1), jnp.float32))


_RA = 512


def _make_agg(F_in, F_g):
    def body(f_ref, g_ref, w_ref, b_ref, out_ref, sq_ref):
        xi = f_ref[...]
        acc = jnp.zeros((_RA, D), jnp.float32)
        for k in range(K):
            xj = g_ref[:, k * F_g:k * F_g + F_in]
            e = jnp.concatenate([xi, xj - xi], axis=1)
            h = jnp.dot(e, w_ref[...],
                        preferred_element_type=jnp.float32) + b_ref[...]
            acc = acc + _lrelu(h)
        out_ref[...] = acc
        sq_ref[...] = jnp.sum(acc * acc, axis=1, keepdims=True)

    return pl.pallas_call(
        body,
        grid=(N // _RA,),
        in_specs=[
            pl.BlockSpec((_RA, F_in), lambda i: (i, 0)),
            pl.BlockSpec((_RA, K * F_g), lambda i: (i, 0)),
            pl.BlockSpec((2 * F_in, D), lambda i: (0, 0)),
            pl.BlockSpec((1, D), lambda i: (0, 0)),
        ],
        out_specs=[
            pl.BlockSpec((_RA, D), lambda i: (i, 0)),
            pl.BlockSpec((_RA, 1), lambda i: (i, 0)),
        ],
        out_shape=[
            jax.ShapeDtypeStruct((N, D), jnp.float32),
            jax.ShapeDtypeStruct((N, 1), jnp.float32),
        ])


_AGG1 = _make_agg(8, 16)
_AGG2 = _make_agg(D, D)


@functools.cache
def _build_sc_gather(Dg):
    mesh = plsc.VectorSubcoreMesh(core_axis_name="c", subcore_axis_name="s")

    @functools.partial(
        pl.kernel, mesh=mesh,
        compiler_params=pltpu.CompilerParams(use_tc_tiling_on_sc=False),
        out_type=jax.ShapeDtypeStruct((_B, Dg), jnp.float32),
        scratch_types=[
            pltpu.VMEM((_GCH, 128), jnp.int32),
            pltpu.VMEM((_BPW, Dg), jnp.float32),
            pltpu.SemaphoreType.DMA,
        ])
    def _sc_gather(table_hbm, idx_hbm, out_hbm, idx_v, rows_v, sem):
        wid = lax.axis_index("s") * _SC_NC + lax.axis_index("c")
        base = wid * _BPW
        pltpu.sync_copy(idx_hbm.at[wid], idx_v)
        handles = []
        for j in range(_GCH):
            handles.append(pltpu.async_copy(
                table_hbm.at[idx_v.at[j]], rows_v.at[pl.ds(j * 128, 128)],
                sem))
        for h in handles:
            h.wait()
        pltpu.sync_copy(rows_v, out_hbm.at[pl.ds(base, _BPW)])

    return _sc_gather


def _gather_rows(table, flat_idx):
    idx3 = flat_idx.reshape(_SC_NW, _GCH, 128)
    return _build_sc_gather(table.shape[1])(table, idx3)


_RH = 512


def _head_body(x1, x2, x3, x4, x5, x6, w1_ref, b1_ref, w2_ref, b2_ref,
               br_ref, pooled_ref):
    t = pl.program_id(0)
    xs = (x1, x2, x3, x4, x5, x6)
    u = b1_ref[...] + jnp.zeros((_RH, 512), jnp.float32)
    for i, xr in enumerate(xs):
        u = u + jnp.dot(xr[...], w1_ref[i * D:(i + 1) * D, :],
                        preferred_element_type=jnp.float32)
    h = jnp.dot(_lrelu(u), w2_ref[...],
                preferred_element_type=jnp.float32) + b2_ref[...]
    br = br_ref[...]
    parts = []
    for g in range(NG):
        hg = jnp.where(br == g, h, -INF)
        parts.append(jnp.max(hg, axis=0, keepdims=True))
    tile_max = jnp.concatenate(parts, axis=0)

    @pl.when(t == 0)
    def _():
        pooled_ref[...] = jnp.full((NG, 512), -np.inf, jnp.float32)

    pooled_ref[...] = jnp.maximum(pooled_ref[...], tile_max)


_HEAD = pl.pallas_call(
    _head_body,
    grid=(N // _RH,),
    in_specs=[pl.BlockSpec((_RH, D), lambda i: (i, 0)) for _ in range(6)] + [
        pl.BlockSpec((6 * D, 512), lambda i: (0, 0)),
        pl.BlockSpec((1, 512), lambda i: (0, 0)),
        pl.BlockSpec((512, 512), lambda i: (0, 0)),
        pl.BlockSpec((1, 512), lambda i: (0, 0)),
        pl.BlockSpec((_RH, 1), lambda i: (i, 0)),
    ],
    out_specs=pl.BlockSpec((NG, 512), lambda i: (0, 0)),
    out_shape=jax.ShapeDtypeStruct((NG, 512), jnp.float32))


def _final_body(pooled_ref, w1_ref, b1_ref, w2_ref, b2_ref, out_ref):
    u = jnp.dot(pooled_ref[...], w1_ref[...],
                preferred_element_type=jnp.float32) + b1_ref[...]
    out_ref[...] = jnp.dot(_lrelu(u), w2_ref[...],
                           preferred_element_type=jnp.float32) + b2_ref[...]


_FINAL = pl.pallas_call(
    _final_body,
    out_shape=jax.ShapeDtypeStruct((NG, K), jnp.float32))


def kernel(x, pos, tq, batch, W1, b1, W2, b2,
           l1W1, l1b1, l1W2, l1b2, mW1, mb1, mW2, mb2):
    batch = batch.astype(jnp.int32)
    xx = jnp.concatenate([tq, x, pos], axis=1)
    xx = jnp.pad(xx, ((0, 0), (0, 3)))
    br = batch.reshape(N, 1)
    bc = batch.reshape(1, N)

    g = jnp.arange(NG, dtype=jnp.int32)
    gstart = jnp.searchsorted(batch, g, side="left").astype(jnp.int32)
    gend = jnp.searchsorted(batch, g, side="right").astype(jnp.int32)
    tidx = jnp.arange(T, dtype=jnp.int32)
    tfirst = batch[tidx * R]
    tlast = batch[tidx * R + (R - 1)]
    clo = (gstart[tfirst] // C).astype(jnp.int32)
    chi = ((gend[tlast] + (C - 1)) // C).astype(jnp.int32)

    W1p = jnp.zeros((16, D), jnp.float32)
    W1p = W1p.at[0:5].set(W1[:5]).at[8:13].set(W1[5:])

    f = xx
    sq = _SQ8(f)
    feats = []
    for i in range(6):
        F = 8 if i == 0 else D
        idx = _KNN[F](clo, chi, f, f, sq, sq.reshape(1, N), br, bc)
        flat = idx.reshape(-1)
        if i == 0:
            table = jnp.pad(f, ((0, 0), (0, 8)))
            G = _gather_rows(table, flat).reshape(N, K * 16)
            f, sq = _AGG1(f, G, W1p, b1.reshape(1, D))
        else:
            G = _gather_rows(f, flat).reshape(N, K * D)
            f, sq = _AGG2(f, G, W2, b2.reshape(1, D))
        feats.append(f)

    pooled = _HEAD(*feats, l1W1, l1b1.reshape(1, 512), l1W2,
                   l1b2.reshape(1, 512), br)
    out = _FINAL(pooled, mW1, mb1.reshape(1, 256), mW2, mb2.reshape(1, K))
    return out
